# baseline (speedup 1.0000x reference)
import jax
import jax.numpy as jnp
from jax.experimental import pallas as pl
from jax.experimental.pallas import tpu as pltpu

_B, _H, _S, _D = 1, 16, 2048, 64
_BLOCK = 64
_NB = _S // _BLOCK
_KEEP = max(1, _NB // 4)
_TEMP = 2.0
_CLAMP_MIN, _CLAMP_MAX = -50.0, 50.0
_SCALE = 1.0 / (_D ** 0.5)
_TQ = 256
_TQR = _TQ // _BLOCK
_NQT = _S // _TQ
_NEG = -1e9

_HI = jax.lax.Precision.HIGHEST


def _attn_kernel(q_ref, k_ref, v_ref, cos_ref, sin_ref, wq_ref, wk_ref,
                 out_ref, gate_ref, bi_ref, kr_ref, gk_ref):
    t = pl.program_id(1)
    f32 = jnp.float32

    mi = jax.lax.broadcasted_iota(jnp.int32, (_D, _D), 0)
    ji = jax.lax.broadcasted_iota(jnp.int32, (_D, _D), 1)
    rot = (mi + _D // 2 == ji).astype(f32) - (mi == ji + _D // 2).astype(f32)

    @pl.when(t == 0)
    def _k_side():
        k = k_ref[0]
        kr = k * cos_ref[...] + jax.lax.dot(k, rot, precision=_HI) * sin_ref[...]
        kr_ref[...] = kr
        pr = jax.lax.broadcasted_iota(jnp.int32, (_NB, _S), 0)
        pc = jax.lax.broadcasted_iota(jnp.int32, (_NB, _S), 1)
        pool_k = (pc // _BLOCK == pr).astype(f32) * (1.0 / _BLOCK)
        kpool = jax.lax.dot(pool_k, kr, precision=_HI)
        gk_ref[...] = jax.lax.dot(kpool, wk_ref[...], preferred_element_type=f32)

    q = q_ref[0]
    cq = cos_ref[pl.ds(t * _TQ, _TQ), :]
    sq = sin_ref[pl.ds(t * _TQ, _TQ), :]
    qr = q * cq + jax.lax.dot(q, rot, precision=_HI) * sq
    qr8 = qr * _SCALE

    qr_ = jax.lax.broadcasted_iota(jnp.int32, (_TQR, _TQ), 0)
    qc_ = jax.lax.broadcasted_iota(jnp.int32, (_TQR, _TQ), 1)
    pool_q = (qc_ // _BLOCK == qr_).astype(f32) * (1.0 / _BLOCK)
    qpool = jax.lax.dot(pool_q, qr, precision=_HI)
    gq = jax.lax.dot(qpool, wq_ref[...], preferred_element_type=f32)
    gl = jax.lax.dot_general(
        gq, gk_ref[...], (((1,), (1,)), ((), ())),
        preferred_element_type=f32) * _SCALE

    rb = t * _TQR + jax.lax.broadcasted_iota(jnp.int32, (_TQR, _NB), 0)
    cb = jax.lax.broadcasted_iota(jnp.int32, (_TQR, _NB), 1)
    bcausal = cb <= rb
    glm = jnp.where(bcausal, gl, _NEG)
    counts = jnp.zeros((_TQR, _NB), f32)
    for m in range(_NB):
        counts = counts + (glm[:, m:m + 1] > glm).astype(f32)
    keep = ((counts < _KEEP) & bcausal) | (cb == rb)
    keep_f = keep.astype(f32)

    er = jax.lax.broadcasted_iota(jnp.int32, (_TQ, _TQR), 0)
    ec = jax.lax.broadcasted_iota(jnp.int32, (_TQ, _TQR), 1)
    expand_q = (er // _BLOCK == ec).astype(f32)
    rowmask = jax.lax.dot(expand_q, keep_f, precision=_HI)

    lane32 = jax.lax.broadcasted_iota(jnp.int32, (_TQ, _NB), 1)
    expand4 = (jax.lax.broadcasted_iota(jnp.int32, (_TQR, _TQ), 1) // _BLOCK
               == jax.lax.broadcasted_iota(jnp.int32, (_TQR, _TQ), 0)).astype(f32)
    ssel_r = jax.lax.broadcasted_iota(jnp.int32, (_NB, _TQR), 0)
    ssel_c = jax.lax.broadcasted_iota(jnp.int32, (_NB, _TQR), 1)

    def _chunk(c, s_c, state):
        m, ssum, acc, cmax = state
        blks = [jnp.max(s_c[:, jj * _BLOCK:(jj + 1) * _BLOCK], axis=1, keepdims=True)
                for jj in range(_TQR)]
        for jj in range(_TQR):
            cmax = jnp.where(lane32 == c * _TQR + jj, blks[jj], cmax)
        ssel = (ssel_r == c * _TQR + ssel_c).astype(f32)
        msel = jax.lax.dot(rowmask, ssel, precision=_HI)
        ckm = jnp.full((_TQ, 1), _NEG, f32)
        for jj in range(_TQR):
            ckm = jnp.maximum(ckm, jnp.where(msel[:, jj:jj + 1] > 0.5, blks[jj], _NEG))
        m_new = jnp.maximum(m, ckm)
        alpha = jnp.exp(m - m_new)
        emask = jax.lax.dot(msel, expand4, precision=_HI)
        e_c = jnp.exp(s_c - m_new) * emask
        ssum = ssum * alpha + jnp.sum(e_c, axis=1, keepdims=True)
        vc = v_ref[0, pl.ds(c * _TQ, _TQ), :]
        acc = acc * alpha + jax.lax.dot(e_c, vc, preferred_element_type=f32)
        return m_new, ssum, acc, cmax

    kd = kr_ref[pl.ds(t * _TQ, _TQ), :]
    s_d = jax.lax.dot_general(qr8, kd, (((1,), (1,)), ((), ())),
                              preferred_element_type=f32)
    row_loc = jax.lax.broadcasted_iota(jnp.int32, (_TQ, _TQ), 0)
    col_loc = jax.lax.broadcasted_iota(jnp.int32, (_TQ, _TQ), 1)
    s_d = jnp.where(row_loc >= col_loc, s_d, _NEG)
    state = (jnp.full((_TQ, 1), _NEG, f32), jnp.zeros((_TQ, 1), f32),
             jnp.zeros((_TQ, _D), f32), jnp.full((_TQ, _NB), _NEG, f32))
    state = _chunk(t, s_d, state)

    def _interior(c, state):
        kc = kr_ref[pl.ds(c * _TQ, _TQ), :]
        s_c = jax.lax.dot_general(qr8, kc, (((1,), (1,)), ((), ())),
                                  preferred_element_type=f32)
        return _chunk(c, s_c, state)

    m, ssum, acc, cmax = jax.lax.fori_loop(0, t, _interior, state)
    out_ref[0] = acc * (1.0 / ssum)

    rsub = jax.lax.broadcasted_iota(jnp.int32, (_TQR, _NB), 0)
    bimp = jnp.full((_TQR, _NB), _NEG, f32)
    for i in range(_TQR):
        rowmax = jnp.max(cmax[i * _BLOCK:(i + 1) * _BLOCK, :], axis=0, keepdims=True)
        bimp = jnp.where(rsub == i, rowmax, bimp)
    bi_ref[pl.ds(t * _TQR, _TQR), :] = bimp

    @pl.when(t == _NQT - 1)
    def _emit_gate():
        x = jnp.clip(bi_ref[...] * (1.0 / _TEMP), _CLAMP_MIN, _CLAMP_MAX)
        ex = jnp.exp(x - jnp.max(x))
        gate_ref[0] = ex / jnp.sum(ex)


def kernel(q, k, v, cos, sin, Wg_q, Wg_k):
    f32 = jnp.float32
    qh = q.reshape(_H, _S, _D)
    kh = k.reshape(_H, _S, _D)
    vh = v.reshape(_H, _S, _D)
    cosh = cos.reshape(_S, _D)
    sinh = sin.reshape(_S, _D)
    out, gate = pl.pallas_call(
        _attn_kernel,
        grid=(_H, _NQT),
        in_specs=[
            pl.BlockSpec((1, _TQ, _D), lambda h, t: (h, t, 0)),
            pl.BlockSpec((1, _S, _D), lambda h, t: (h, 0, 0)),
            pl.BlockSpec((1, _S, _D), lambda h, t: (h, 0, 0)),
            pl.BlockSpec((_S, _D), lambda h, t: (0, 0)),
            pl.BlockSpec((_S, _D), lambda h, t: (0, 0)),
            pl.BlockSpec((_D, _D), lambda h, t: (0, 0)),
            pl.BlockSpec((_D, _D), lambda h, t: (0, 0)),
        ],
        out_specs=[
            pl.BlockSpec((1, _TQ, _D), lambda h, t: (h, t, 0)),
            pl.BlockSpec((1, _NB, _NB), lambda h, t: (h, 0, 0)),
        ],
        out_shape=[
            jax.ShapeDtypeStruct((_H, _S, _D), f32),
            jax.ShapeDtypeStruct((_H, _NB, _NB), f32),
        ],
        scratch_shapes=[
            pltpu.VMEM((_NB, _NB), f32),
            pltpu.VMEM((_S, _D), f32),
            pltpu.VMEM((_NB, _D), f32),
        ],
    )(qh, kh, vh, cosh, sinh, Wg_q, Wg_k)
    return out.reshape(_B, _H, _S, _D), gate.reshape(_B, _H, _NB, _NB)


# speedup vs baseline: 1.5453x; 1.5453x over previous
import jax
import jax.numpy as jnp
from jax.experimental import pallas as pl
from jax.experimental.pallas import tpu as pltpu

_B, _H, _S, _D = 1, 16, 2048, 64
_BLOCK = 64
_NB = _S // _BLOCK
_KEEP = max(1, _NB // 4)
_TEMP = 2.0
_CLAMP_MIN, _CLAMP_MAX = -50.0, 50.0
_SCALE = 1.0 / (_D ** 0.5)
_TQ = 512
_TQR = _TQ // _BLOCK
_NQT = _S // _TQ
_NEG = -1e9

_HI = jax.lax.Precision.HIGHEST


def _attn_kernel(q_ref, k_ref, v_ref, cos_ref, sin_ref, wq_ref, wk_ref,
                 out_ref, gate_ref, bi_ref, kr_ref, gk_ref):
    t = pl.program_id(1)
    f32 = jnp.float32

    mi = jax.lax.broadcasted_iota(jnp.int32, (_D, _D), 0)
    ji = jax.lax.broadcasted_iota(jnp.int32, (_D, _D), 1)
    rot = (mi + _D // 2 == ji).astype(f32) - (mi == ji + _D // 2).astype(f32)

    @pl.when(t == 0)
    def _k_side():
        k = k_ref[0]
        kr = k * cos_ref[...] + jax.lax.dot(k, rot, precision=_HI) * sin_ref[...]
        kr_ref[...] = kr
        pr = jax.lax.broadcasted_iota(jnp.int32, (_NB, _S), 0)
        pc = jax.lax.broadcasted_iota(jnp.int32, (_NB, _S), 1)
        pool_k = (pc // _BLOCK == pr).astype(f32) * (1.0 / _BLOCK)
        kpool = jax.lax.dot(pool_k, kr, precision=_HI)
        gk_ref[...] = jax.lax.dot(kpool, wk_ref[...], preferred_element_type=f32)

    kr = kr_ref[...]
    q = q_ref[0]
    cq = cos_ref[pl.ds(t * _TQ, _TQ), :]
    sq = sin_ref[pl.ds(t * _TQ, _TQ), :]
    qr = q * cq + jax.lax.dot(q, rot, precision=_HI) * sq

    scores = jax.lax.dot_general(
        qr * _SCALE, kr, (((1,), (1,)), ((), ())),
        preferred_element_type=f32)

    row = t * _TQ + jax.lax.broadcasted_iota(jnp.int32, (_TQ, _S), 0)
    col = jax.lax.broadcasted_iota(jnp.int32, (_TQ, _S), 1)
    sc = jnp.where(row >= col, scores, _NEG)

    lane = jax.lax.broadcasted_iota(jnp.int32, (_TQ, _NB), 1)
    cmax = jnp.full((_TQ, _NB), _NEG, f32)
    for j in range(_NB):
        blkmax = jnp.max(sc[:, j * _BLOCK:(j + 1) * _BLOCK], axis=1, keepdims=True)
        cmax = jnp.where(lane == j, blkmax, cmax)
    rsub = jax.lax.broadcasted_iota(jnp.int32, (_TQR, _NB), 0)
    bimp = jnp.full((_TQR, _NB), _NEG, f32)
    for i in range(_TQR):
        rowmax = jnp.max(cmax[i * _BLOCK:(i + 1) * _BLOCK, :], axis=0, keepdims=True)
        bimp = jnp.where(rsub == i, rowmax, bimp)
    bi_ref[pl.ds(t * _TQR, _TQR), :] = bimp

    qr_ = jax.lax.broadcasted_iota(jnp.int32, (_TQR, _TQ), 0)
    qc_ = jax.lax.broadcasted_iota(jnp.int32, (_TQR, _TQ), 1)
    pool_q = (qc_ // _BLOCK == qr_).astype(f32) * (1.0 / _BLOCK)
    qpool = jax.lax.dot(pool_q, qr, precision=_HI)
    gq = jax.lax.dot(qpool, wq_ref[...], preferred_element_type=f32)
    gl = jax.lax.dot_general(
        gq, gk_ref[...], (((1,), (1,)), ((), ())),
        preferred_element_type=f32) * _SCALE

    rb = t * _TQR + jax.lax.broadcasted_iota(jnp.int32, (_TQR, _NB), 0)
    cb = jax.lax.broadcasted_iota(jnp.int32, (_TQR, _NB), 1)
    bcausal = cb <= rb
    glm = jnp.where(bcausal, gl, _NEG)
    counts = jnp.zeros((_TQR, _NB), f32)
    for m in range(_NB):
        counts = counts + (glm[:, m:m + 1] > glm).astype(f32)
    keep = ((counts < _KEEP) & bcausal) | (cb == rb)
    keep_f = keep.astype(f32)

    er = jax.lax.broadcasted_iota(jnp.int32, (_TQ, _TQR), 0)
    ec = jax.lax.broadcasted_iota(jnp.int32, (_TQ, _TQR), 1)
    expand_q = (er // _BLOCK == ec).astype(f32)
    rowmask = jax.lax.dot(expand_q, keep_f, precision=_HI)

    mrow = jnp.max(jnp.where(rowmask > 0.5, cmax, _NEG), axis=1, keepdims=True)

    pr2 = jax.lax.broadcasted_iota(jnp.int32, (_NB, _S), 0)
    pc2 = jax.lax.broadcasted_iota(jnp.int32, (_NB, _S), 1)
    expand_k = (pc2 // _BLOCK == pr2).astype(f32)
    fullmask = jax.lax.dot(rowmask, expand_k, precision=_HI)

    e = jnp.exp(sc - mrow) * fullmask
    ssum = jnp.sum(e, axis=1, keepdims=True)
    acc = jax.lax.dot(e, v_ref[0], preferred_element_type=f32)
    out_ref[0] = acc * (1.0 / ssum)

    @pl.when(t == _NQT - 1)
    def _emit_gate():
        x = jnp.clip(bi_ref[...] * (1.0 / _TEMP), _CLAMP_MIN, _CLAMP_MAX)
        ex = jnp.exp(x - jnp.max(x))
        gate_ref[0] = ex / jnp.sum(ex)


def kernel(q, k, v, cos, sin, Wg_q, Wg_k):
    f32 = jnp.float32
    qh = q.reshape(_H, _S, _D)
    kh = k.reshape(_H, _S, _D)
    vh = v.reshape(_H, _S, _D)
    cosh = cos.reshape(_S, _D)
    sinh = sin.reshape(_S, _D)
    out, gate = pl.pallas_call(
        _attn_kernel,
        grid=(_H, _NQT),
        in_specs=[
            pl.BlockSpec((1, _TQ, _D), lambda h, t: (h, t, 0)),
            pl.BlockSpec((1, _S, _D), lambda h, t: (h, 0, 0)),
            pl.BlockSpec((1, _S, _D), lambda h, t: (h, 0, 0)),
            pl.BlockSpec((_S, _D), lambda h, t: (0, 0)),
            pl.BlockSpec((_S, _D), lambda h, t: (0, 0)),
            pl.BlockSpec((_D, _D), lambda h, t: (0, 0)),
            pl.BlockSpec((_D, _D), lambda h, t: (0, 0)),
        ],
        out_specs=[
            pl.BlockSpec((1, _TQ, _D), lambda h, t: (h, t, 0)),
            pl.BlockSpec((1, _NB, _NB), lambda h, t: (h, 0, 0)),
        ],
        out_shape=[
            jax.ShapeDtypeStruct((_H, _S, _D), f32),
            jax.ShapeDtypeStruct((_H, _NB, _NB), f32),
        ],
        scratch_shapes=[
            pltpu.VMEM((_NB, _NB), f32),
            pltpu.VMEM((_S, _D), f32),
            pltpu.VMEM((_NB, _D), f32),
        ],
    )(qh, kh, vh, cosh, sinh, Wg_q, Wg_k)
    return out.reshape(_B, _H, _S, _D), gate.reshape(_B, _H, _NB, _NB)
